# Initial kernel scaffold; baseline (speedup 1.0000x reference)
import jax
import jax.numpy as jnp
from jax.experimental import pallas as pl


def kernel(x, edge_index, W1, b1, W2, b2):
    raise NotImplementedError("write your pallas kernel here")


# speedup vs baseline: 19.6809x; 19.6809x over previous
import functools

import jax
import jax.numpy as jnp
from jax import lax
from jax.experimental import pallas as pl
from jax.experimental.pallas import tpu as pltpu
from jax.experimental.pallas import tpu_sc as plsc

N = 10000
D = 128
NPAD = 10240
E = 320000
NW = 32
CHUNK = 128
CPW = 80
EPW = CPW * CHUNK
EPAD = NW * EPW
ROWS_PT = NPAD // 16
BLK = 512
GRID = NPAD // BLK

_mesh = functools.partial(
    plsc.VectorSubcoreMesh, core_axis_name="c", subcore_axis_name="s")


def _deg_body(dst_hbm, onehot_hbm, z16_hbm, out_hbm, dstv, onev, acc):
    c = lax.axis_index("c")
    s = lax.axis_index("s")
    wid = c * 16 + s
    pltpu.sync_copy(z16_hbm, acc.at[pl.ds(s * ROWS_PT, ROWS_PT)])
    pltpu.sync_copy(dst_hbm.at[pl.ds(wid * CPW, CPW)], dstv)
    pltpu.sync_copy(onehot_hbm, onev)
    plsc.subcore_barrier()

    def body(j, carry):
        pltpu.sync_copy(onev, acc.at[dstv.at[j]], add=True)
        return carry

    lax.fori_loop(0, CPW, body, 0)
    plsc.subcore_barrier()
    pltpu.sync_copy(acc.at[pl.ds(s * ROWS_PT, ROWS_PT)],
                    out_hbm.at[c, pl.ds(s * ROWS_PT, ROWS_PT)])


def _deg_sc(dst2d, onehot, z16):
    return pl.kernel(
        _deg_body,
        out_type=jax.ShapeDtypeStruct((2, NPAD, 16), jnp.float32),
        mesh=_mesh(),
        scratch_types=[
            pltpu.VMEM((CPW, CHUNK), jnp.int32),
            pltpu.VMEM((CHUNK, 16), jnp.float32),
            pltpu.VMEM_SHARED((NPAD, 16), jnp.float32),
        ],
    )(dst2d, onehot, z16)


def _edge_body(hs_hbm, src_hbm, dst_hbm, z128_hbm, out_hbm,
               srcv, dstv, gbuf, acc):
    c = lax.axis_index("c")
    s = lax.axis_index("s")
    wid = c * 16 + s
    pltpu.sync_copy(z128_hbm, acc.at[pl.ds(s * ROWS_PT, ROWS_PT)])
    pltpu.sync_copy(src_hbm.at[pl.ds(wid * CPW, CPW)], srcv)
    pltpu.sync_copy(dst_hbm.at[pl.ds(wid * CPW, CPW)], dstv)
    plsc.subcore_barrier()

    def body(j, carry):
        pltpu.sync_copy(hs_hbm.at[srcv.at[j]], gbuf)
        pltpu.sync_copy(gbuf, acc.at[dstv.at[j]], add=True)
        return carry

    lax.fori_loop(0, CPW, body, 0)
    plsc.subcore_barrier()
    pltpu.sync_copy(acc.at[pl.ds(s * ROWS_PT, ROWS_PT)],
                    out_hbm.at[c, pl.ds(s * ROWS_PT, ROWS_PT)])


def _edge_sc(hs, src2d, dst2d, z128):
    return pl.kernel(
        _edge_body,
        out_type=jax.ShapeDtypeStruct((2, NPAD, D), jnp.float32),
        mesh=_mesh(),
        scratch_types=[
            pltpu.VMEM((CPW, CHUNK), jnp.int32),
            pltpu.VMEM((CPW, CHUNK), jnp.int32),
            pltpu.VMEM((CHUNK, D), jnp.float32),
            pltpu.VMEM_SHARED((NPAD, D), jnp.float32),
        ],
    )(hs, src2d, dst2d, z128)


def _prep_body(x_ref, w_ref, d0_ref, d1_ref, hs_ref, di_ref):
    deg = jnp.sum(d0_ref[...] + d1_ref[...], axis=1, keepdims=True) + 1.0
    di = lax.rsqrt(deg)
    hs_ref[...] = jnp.dot(x_ref[...], w_ref[...],
                          preferred_element_type=jnp.float32) * di
    di_ref[...] = di


def _prep_tc(x, w1, d0, d1):
    return pl.pallas_call(
        _prep_body,
        grid=(GRID,),
        in_specs=[
            pl.BlockSpec((BLK, D), lambda i: (i, 0)),
            pl.BlockSpec((D, D), lambda i: (0, 0)),
            pl.BlockSpec((BLK, 16), lambda i: (i, 0)),
            pl.BlockSpec((BLK, 16), lambda i: (i, 0)),
        ],
        out_specs=[
            pl.BlockSpec((BLK, D), lambda i: (i, 0)),
            pl.BlockSpec((BLK, 1), lambda i: (i, 0)),
        ],
        out_shape=[
            jax.ShapeDtypeStruct((NPAD, D), jnp.float32),
            jax.ShapeDtypeStruct((NPAD, 1), jnp.float32),
        ],
    )(x, w1, d0, d1)


def _mid_body(a0_ref, a1_ref, hs_ref, di_ref, b_ref, w_ref, out_ref):
    di = di_ref[...]
    h = di * (a0_ref[...] + a1_ref[...] + hs_ref[...]) + b_ref[...]
    h = jnp.maximum(h, 0.0)
    out_ref[...] = jnp.dot(h, w_ref[...],
                           preferred_element_type=jnp.float32) * di


def _mid_tc(a0, a1, hs, di, b1, w2):
    return pl.pallas_call(
        _mid_body,
        grid=(GRID,),
        in_specs=[
            pl.BlockSpec((BLK, D), lambda i: (i, 0)),
            pl.BlockSpec((BLK, D), lambda i: (i, 0)),
            pl.BlockSpec((BLK, D), lambda i: (i, 0)),
            pl.BlockSpec((BLK, 1), lambda i: (i, 0)),
            pl.BlockSpec((1, D), lambda i: (0, 0)),
            pl.BlockSpec((D, D), lambda i: (0, 0)),
        ],
        out_specs=pl.BlockSpec((BLK, D), lambda i: (i, 0)),
        out_shape=jax.ShapeDtypeStruct((NPAD, D), jnp.float32),
    )(a0, a1, hs, di, b1, w2)


def _final_body(a0_ref, a1_ref, hs_ref, di_ref, b_ref, h_ref, p_ref):
    i = pl.program_id(0)
    h = di_ref[...] * (a0_ref[...] + a1_ref[...] + hs_ref[...]) + b_ref[...]
    h = jnp.maximum(h, 0.0)
    h_ref[...] = h
    row = lax.broadcasted_iota(jnp.int32, (BLK, D), 0) + i * BLK
    hm = jnp.where(row < N, h, -jnp.inf)
    bm = jnp.max(hm, axis=0, keepdims=True)

    @pl.when(i == 0)
    def _():
        p_ref[...] = bm

    @pl.when(i > 0)
    def _():
        p_ref[...] = jnp.maximum(p_ref[...], bm)


def _final_tc(a0, a1, hs, di, b2):
    return pl.pallas_call(
        _final_body,
        grid=(GRID,),
        in_specs=[
            pl.BlockSpec((BLK, D), lambda i: (i, 0)),
            pl.BlockSpec((BLK, D), lambda i: (i, 0)),
            pl.BlockSpec((BLK, D), lambda i: (i, 0)),
            pl.BlockSpec((BLK, 1), lambda i: (i, 0)),
            pl.BlockSpec((1, D), lambda i: (0, 0)),
        ],
        out_specs=[
            pl.BlockSpec((BLK, D), lambda i: (i, 0)),
            pl.BlockSpec((1, D), lambda i: (0, 0)),
        ],
        out_shape=[
            jax.ShapeDtypeStruct((NPAD, D), jnp.float32),
            jax.ShapeDtypeStruct((1, D), jnp.float32),
        ],
    )(a0, a1, hs, di, b2)


def kernel(x, edge_index, W1, b1, W2, b2):
    src = edge_index[0].astype(jnp.int32)
    dst = edge_index[1].astype(jnp.int32)
    pad = (N + (jnp.arange(EPAD - E, dtype=jnp.int32) % (NPAD - N)))
    src2d = jnp.concatenate([src, pad]).reshape(NW * CPW, CHUNK)
    dst2d = jnp.concatenate([dst, pad]).reshape(NW * CPW, CHUNK)
    xp = jnp.zeros((NPAD, D), jnp.float32).at[:N].set(x)

    onehot = jnp.zeros((CHUNK, 16), jnp.float32).at[:, 0].set(1.0)
    z16 = jnp.zeros((ROWS_PT, 16), jnp.float32)
    z128 = jnp.zeros((ROWS_PT, D), jnp.float32)
    b1r = b1.reshape(1, D)
    b2r = b2.reshape(1, D)

    degs = _deg_sc(dst2d, onehot, z16)
    hs1, dinv = _prep_tc(xp, W1, degs[0], degs[1])
    acc1 = _edge_sc(hs1, src2d, dst2d, z128)
    hs2 = _mid_tc(acc1[0], acc1[1], hs1, dinv, b1r, W2)
    acc2 = _edge_sc(hs2, src2d, dst2d, z128)
    h2, p = _final_tc(acc2[0], acc2[1], hs2, dinv, b2r)
    return (h2[:N], p)


# speedup vs baseline: 23.3120x; 1.1845x over previous
import functools

import jax
import jax.numpy as jnp
from jax import lax
from jax.experimental import pallas as pl
from jax.experimental.pallas import tpu as pltpu
from jax.experimental.pallas import tpu_sc as plsc

N = 10000
D = 128
NPAD = 10240
E = 320000
NW = 32
CHUNK = 128
CPW = 80
EPW = CPW * CHUNK
EPAD = NW * EPW
ROWS_PT = NPAD // 16
BLK = 512
GRID = NPAD // BLK

_mesh = functools.partial(
    plsc.VectorSubcoreMesh, core_axis_name="c", subcore_axis_name="s")


def _deg_body(dst_hbm, onehot_hbm, z16_hbm, out_hbm, dstv, onev, acc):
    c = lax.axis_index("c")
    s = lax.axis_index("s")
    wid = c * 16 + s
    pltpu.sync_copy(z16_hbm, acc.at[pl.ds(s * ROWS_PT, ROWS_PT)])
    pltpu.sync_copy(dst_hbm.at[pl.ds(wid * CPW, CPW)], dstv)
    pltpu.sync_copy(onehot_hbm, onev)
    plsc.subcore_barrier()

    def body(j, carry):
        pltpu.sync_copy(onev, acc.at[dstv.at[j]], add=True)
        return carry

    lax.fori_loop(0, CPW, body, 0)
    plsc.subcore_barrier()
    pltpu.sync_copy(acc.at[pl.ds(s * ROWS_PT, ROWS_PT)],
                    out_hbm.at[c, pl.ds(s * ROWS_PT, ROWS_PT)])


def _deg_sc(dst2d, onehot, z16):
    return pl.kernel(
        _deg_body,
        out_type=jax.ShapeDtypeStruct((2, NPAD, 16), jnp.float32),
        mesh=_mesh(),
        scratch_types=[
            pltpu.VMEM((CPW, CHUNK), jnp.int32),
            pltpu.VMEM((CHUNK, 16), jnp.float32),
            pltpu.VMEM_SHARED((NPAD, 16), jnp.float32),
        ],
    )(dst2d, onehot, z16)


GRP = 16
NGRP = CPW // GRP


def _edge_body(hs_hbm, src_hbm, dst_hbm, z128_hbm, out_hbm,
               si, di, ga, gb, sa, sb, sia, sib, acc):
    c = lax.axis_index("c")
    s = lax.axis_index("s")
    wid = c * 16 + s
    base = wid * CPW
    pltpu.sync_copy(z128_hbm, acc.at[pl.ds(s * ROWS_PT, ROWS_PT)])

    pltpu.async_copy(src_hbm.at[pl.ds(base, GRP)], si.at[0], sia)
    pltpu.async_copy(dst_hbm.at[pl.ds(base, GRP)], di.at[0], sia)
    plsc.subcore_barrier()

    for g in range(NGRP):
        p, q = g % 2, (g + 1) % 2
        sem = sia if p == 0 else sib
        gb_ = base + g * GRP
        pltpu.make_async_copy(src_hbm.at[pl.ds(gb_, GRP)], si.at[p], sem).wait()
        pltpu.make_async_copy(dst_hbm.at[pl.ds(gb_, GRP)], di.at[p], sem).wait()
        if g + 1 < NGRP:
            nsem = sia if q == 0 else sib
            nb = base + (g + 1) * GRP
            pltpu.async_copy(src_hbm.at[pl.ds(nb, GRP)], si.at[q], nsem)
            pltpu.async_copy(dst_hbm.at[pl.ds(nb, GRP)], di.at[q], nsem)
        srcv, dstv = si.at[p], di.at[p]

        pltpu.async_copy(hs_hbm.at[srcv.at[0]], ga, sa)

        def body(i, carry, srcv=srcv, dstv=dstv):
            j = 2 * i
            pltpu.make_async_copy(hs_hbm.at[srcv.at[j]], ga, sa).wait()
            pltpu.async_copy(hs_hbm.at[srcv.at[j + 1]], gb, sb)
            pltpu.sync_copy(ga, acc.at[dstv.at[j]], add=True)
            pltpu.make_async_copy(hs_hbm.at[srcv.at[j + 1]], gb, sb).wait()
            j2 = jnp.minimum(j + 2, GRP - 1)
            pltpu.async_copy(hs_hbm.at[srcv.at[j2]], ga, sa)
            pltpu.sync_copy(gb, acc.at[dstv.at[j + 1]], add=True)
            return carry

        lax.fori_loop(0, GRP // 2, body, 0)
        pltpu.make_async_copy(hs_hbm.at[srcv.at[0]], ga, sa).wait()

    plsc.subcore_barrier()
    pltpu.sync_copy(acc.at[pl.ds(s * ROWS_PT, ROWS_PT)],
                    out_hbm.at[c, pl.ds(s * ROWS_PT, ROWS_PT)])


def _edge_sc(hs, src2d, dst2d, z128):
    return pl.kernel(
        _edge_body,
        out_type=jax.ShapeDtypeStruct((2, NPAD, D), jnp.float32),
        mesh=_mesh(),
        scratch_types=[
            pltpu.VMEM((2, GRP, CHUNK), jnp.int32),
            pltpu.VMEM((2, GRP, CHUNK), jnp.int32),
            pltpu.VMEM((CHUNK, D), jnp.float32),
            pltpu.VMEM((CHUNK, D), jnp.float32),
            pltpu.SemaphoreType.DMA,
            pltpu.SemaphoreType.DMA,
            pltpu.SemaphoreType.DMA,
            pltpu.SemaphoreType.DMA,
            pltpu.VMEM_SHARED((NPAD, D), jnp.float32),
        ],
    )(hs, src2d, dst2d, z128)


def _prep_body(x_ref, w_ref, d0_ref, d1_ref, hs_ref, di_ref):
    deg = jnp.sum(d0_ref[...] + d1_ref[...], axis=1, keepdims=True) + 1.0
    di = lax.rsqrt(deg)
    hs_ref[...] = jnp.dot(x_ref[...], w_ref[...],
                          preferred_element_type=jnp.float32) * di
    di_ref[...] = di


def _prep_tc(x, w1, d0, d1):
    return pl.pallas_call(
        _prep_body,
        grid=(GRID,),
        in_specs=[
            pl.BlockSpec((BLK, D), lambda i: (i, 0)),
            pl.BlockSpec((D, D), lambda i: (0, 0)),
            pl.BlockSpec((BLK, 16), lambda i: (i, 0)),
            pl.BlockSpec((BLK, 16), lambda i: (i, 0)),
        ],
        out_specs=[
            pl.BlockSpec((BLK, D), lambda i: (i, 0)),
            pl.BlockSpec((BLK, 1), lambda i: (i, 0)),
        ],
        out_shape=[
            jax.ShapeDtypeStruct((NPAD, D), jnp.float32),
            jax.ShapeDtypeStruct((NPAD, 1), jnp.float32),
        ],
    )(x, w1, d0, d1)


def _mid_body(a0_ref, a1_ref, hs_ref, di_ref, b_ref, w_ref, out_ref):
    di = di_ref[...]
    h = di * (a0_ref[...] + a1_ref[...] + hs_ref[...]) + b_ref[...]
    h = jnp.maximum(h, 0.0)
    out_ref[...] = jnp.dot(h, w_ref[...],
                           preferred_element_type=jnp.float32) * di


def _mid_tc(a0, a1, hs, di, b1, w2):
    return pl.pallas_call(
        _mid_body,
        grid=(GRID,),
        in_specs=[
            pl.BlockSpec((BLK, D), lambda i: (i, 0)),
            pl.BlockSpec((BLK, D), lambda i: (i, 0)),
            pl.BlockSpec((BLK, D), lambda i: (i, 0)),
            pl.BlockSpec((BLK, 1), lambda i: (i, 0)),
            pl.BlockSpec((1, D), lambda i: (0, 0)),
            pl.BlockSpec((D, D), lambda i: (0, 0)),
        ],
        out_specs=pl.BlockSpec((BLK, D), lambda i: (i, 0)),
        out_shape=jax.ShapeDtypeStruct((NPAD, D), jnp.float32),
    )(a0, a1, hs, di, b1, w2)


def _final_body(a0_ref, a1_ref, hs_ref, di_ref, b_ref, h_ref, p_ref):
    i = pl.program_id(0)
    h = di_ref[...] * (a0_ref[...] + a1_ref[...] + hs_ref[...]) + b_ref[...]
    h = jnp.maximum(h, 0.0)
    h_ref[...] = h
    row = lax.broadcasted_iota(jnp.int32, (BLK, D), 0) + i * BLK
    hm = jnp.where(row < N, h, -jnp.inf)
    bm = jnp.max(hm, axis=0, keepdims=True)

    @pl.when(i == 0)
    def _():
        p_ref[...] = bm

    @pl.when(i > 0)
    def _():
        p_ref[...] = jnp.maximum(p_ref[...], bm)


def _final_tc(a0, a1, hs, di, b2):
    return pl.pallas_call(
        _final_body,
        grid=(GRID,),
        in_specs=[
            pl.BlockSpec((BLK, D), lambda i: (i, 0)),
            pl.BlockSpec((BLK, D), lambda i: (i, 0)),
            pl.BlockSpec((BLK, D), lambda i: (i, 0)),
            pl.BlockSpec((BLK, 1), lambda i: (i, 0)),
            pl.BlockSpec((1, D), lambda i: (0, 0)),
        ],
        out_specs=[
            pl.BlockSpec((BLK, D), lambda i: (i, 0)),
            pl.BlockSpec((1, D), lambda i: (0, 0)),
        ],
        out_shape=[
            jax.ShapeDtypeStruct((NPAD, D), jnp.float32),
            jax.ShapeDtypeStruct((1, D), jnp.float32),
        ],
    )(a0, a1, hs, di, b2)


def kernel(x, edge_index, W1, b1, W2, b2):
    src = edge_index[0].astype(jnp.int32)
    dst = edge_index[1].astype(jnp.int32)
    pad = (N + (jnp.arange(EPAD - E, dtype=jnp.int32) % (NPAD - N)))
    src2d = jnp.concatenate([src, pad]).reshape(NW * CPW, CHUNK)
    dst2d = jnp.concatenate([dst, pad]).reshape(NW * CPW, CHUNK)
    xp = jnp.zeros((NPAD, D), jnp.float32).at[:N].set(x)

    onehot = jnp.zeros((CHUNK, 16), jnp.float32).at[:, 0].set(1.0)
    z16 = jnp.zeros((ROWS_PT, 16), jnp.float32)
    z128 = jnp.zeros((ROWS_PT, D), jnp.float32)
    b1r = b1.reshape(1, D)
    b2r = b2.reshape(1, D)

    degs = _deg_sc(dst2d, onehot, z16)
    hs1, dinv = _prep_tc(xp, W1, degs[0], degs[1])
    acc1 = _edge_sc(hs1, src2d, dst2d, z128)
    hs2 = _mid_tc(acc1[0], acc1[1], hs1, dinv, b1r, W2)
    acc2 = _edge_sc(hs2, src2d, dst2d, z128)
    h2, p = _final_tc(acc2[0], acc2[1], hs2, dinv, b2r)
    return (h2[:N], p)
